# Initial kernel scaffold; baseline (speedup 1.0000x reference)
import jax
import jax.numpy as jnp
from jax.experimental import pallas as pl


def kernel(x, edge_attr, senders, receivers, W_en, b_en, W_ee, b_ee, We1, be1, We2, be2, Wn1, bn1, Wn2, bn2, Wg1, bg1, Wg2, bg2, Wg3, bg3):
    raise NotImplementedError("write your pallas kernel here")


# speedup vs baseline: 2.7602x; 2.7602x over previous
import functools

import jax
import jax.numpy as jnp
from jax import lax
from jax.experimental import pallas as pl
from jax.experimental.pallas import tpu as pltpu
from jax.experimental.pallas import tpu_sc as plsc

_N = 10000
_E = 160000
_LAT = 128
_H1 = 256
_H2 = 128

_NC, _NS = 2, 16
_NW = _NC * _NS

_NB = 10
_NBLK = _N // _NB
_EBLK = 2000
_EB = _E // _EBLK

_CG = 200
_CS = 200


def _prep_body(x_ref, wen_ref, ben_ref, wb_ref, wc_ref, nodes_ref, s_ref, r_ref):
    nb = jnp.dot(x_ref[...], wen_ref[...], preferred_element_type=jnp.float32)
    nb = nb + ben_ref[...]
    nodes_ref[...] = nb
    s_ref[...] = jnp.dot(nb, wb_ref[...], preferred_element_type=jnp.float32)
    r_ref[...] = jnp.dot(nb, wc_ref[...], preferred_element_type=jnp.float32)


def _prep(x, W_en, b_en, We1b, We1c):
    full = lambda shape: pl.BlockSpec(shape, lambda i: (0, 0))
    return pl.pallas_call(
        _prep_body,
        grid=(_NB,),
        in_specs=[
            pl.BlockSpec((_NBLK, _LAT), lambda i: (i, 0)),
            full((_LAT, _LAT)),
            full((1, _LAT)),
            full((_LAT, _H1)),
            full((_LAT, _H1)),
        ],
        out_specs=[
            pl.BlockSpec((_NBLK, _LAT), lambda i: (i, 0)),
            pl.BlockSpec((_NBLK, _H1), lambda i: (i, 0)),
            pl.BlockSpec((_NBLK, _H1), lambda i: (i, 0)),
        ],
        out_shape=[
            jax.ShapeDtypeStruct((_N, _LAT), jnp.float32),
            jax.ShapeDtypeStruct((_N, _H1), jnp.float32),
            jax.ShapeDtypeStruct((_N, _H1), jnp.float32),
        ],
    )(x, W_en, b_en.reshape(1, _LAT), We1b, We1c)


def _gather_body(s_hbm, r_hbm, snd_hbm, rcv_hbm, g_hbm,
                 idxs_v, idxr_v, bufs_v, bufr_v, sem1, sem2):
    wid = lax.axis_index("s") * _NC + lax.axis_index("c")
    per_w = _E // _NW
    base = wid * per_w

    def chunk(i, carry):
        off = base + i * _CG
        pltpu.sync_copy(snd_hbm.at[pl.ds(off, _CG)], idxs_v)
        pltpu.sync_copy(rcv_hbm.at[pl.ds(off, _CG)], idxr_v)
        cs = pltpu.async_copy(s_hbm.at[idxs_v], bufs_v, sem1)
        cr = pltpu.async_copy(r_hbm.at[idxr_v], bufr_v, sem2)
        cs.wait()
        cr.wait()

        def row(rr, c2):
            for l in range(_H1 // 16):
                sl = pl.ds(l * 16, 16)
                bufs_v[rr, sl] = bufs_v[rr, sl] + bufr_v[rr, sl]
            return c2

        lax.fori_loop(0, _CG, row, 0)
        pltpu.sync_copy(bufs_v, g_hbm.at[pl.ds(off, _CG)])
        return carry

    lax.fori_loop(0, per_w // _CG, chunk, 0)


def _gather(S, R, senders, receivers):
    mesh = plsc.VectorSubcoreMesh(core_axis_name="c", subcore_axis_name="s",
                                  num_cores=_NC, num_subcores=_NS)
    kfn = pl.kernel(
        _gather_body,
        out_type=jax.ShapeDtypeStruct((_E, _H1), jnp.float32),
        mesh=mesh,
        scratch_types=[
            pltpu.VMEM((_CG,), jnp.int32),
            pltpu.VMEM((_CG,), jnp.int32),
            pltpu.VMEM((_CG, _H1), jnp.float32),
            pltpu.VMEM((_CG, _H1), jnp.float32),
            pltpu.SemaphoreType.DMA,
            pltpu.SemaphoreType.DMA,
        ],
    )
    return kfn(S, R, senders, receivers)


def _edge_body(g_ref, ea_ref, wee_ref, bee_ref, we1a_ref, be1_ref,
               we2_ref, be2_ref, out_ref):
    wea = jnp.dot(wee_ref[...], we1a_ref[...], preferred_element_type=jnp.float32)
    bias = jnp.dot(bee_ref[...], we1a_ref[...], preferred_element_type=jnp.float32)
    bias = bias + be1_ref[...]
    h = g_ref[...] + jnp.dot(ea_ref[...], wea, preferred_element_type=jnp.float32)
    h = jnp.maximum(h + bias, 0.0)
    out_ref[...] = jnp.dot(h, we2_ref[...], preferred_element_type=jnp.float32) + be2_ref[...]


def _edge_mlp(G, edge_attr, W_ee, b_ee, We1a, be1, We2, be2):
    full = lambda shape: pl.BlockSpec(shape, lambda i: (0, 0))
    return pl.pallas_call(
        _edge_body,
        grid=(_EB,),
        in_specs=[
            pl.BlockSpec((_EBLK, _H1), lambda i: (i, 0)),
            pl.BlockSpec((_EBLK, 16), lambda i: (i, 0)),
            full((16, _LAT)),
            full((1, _LAT)),
            full((_LAT, _H1)),
            full((1, _H1)),
            full((_H1, _H2)),
            full((1, _H2)),
        ],
        out_specs=pl.BlockSpec((_EBLK, _H2), lambda i: (i, 0)),
        out_shape=jax.ShapeDtypeStruct((_E, _H2), jnp.float32),
    )(G, edge_attr, W_ee, b_ee.reshape(1, _LAT), We1a, be1.reshape(1, _H1),
      We2, be2.reshape(1, _H2))


_NPT = 632
_NPAD = _NS * _NPT


def _scatter_body(en_hbm, idx2_hbm, out_hbm, acc_sh, ebuf, idx_v, sem):
    c = lax.axis_index("c")
    s = lax.axis_index("s")

    def zrow(rr, carry):
        for l in range(_H2 // 16):
            ebuf[rr, pl.ds(l * 16, 16)] = jnp.zeros((16,), jnp.float32)
        return carry

    lax.fori_loop(0, _CS, zrow, 0)
    for k in range(_NPT // _CS):
        pltpu.sync_copy(ebuf, acc_sh.at[pl.ds(s * _NPT + k * _CS, _CS)])
    rem = _NPT % _CS
    if rem:
        pltpu.sync_copy(ebuf.at[pl.ds(0, rem)],
                        acc_sh.at[pl.ds(s * _NPT + (_NPT // _CS) * _CS, rem)])
    plsc.subcore_barrier()

    per_tile = _E // _NS

    def chunk(i, carry):
        off = s * per_tile + i * _CS
        pltpu.sync_copy(idx2_hbm.at[pl.ds(c * _E + off, _CS)], idx_v)
        pltpu.sync_copy(en_hbm.at[pl.ds(off, _CS)], ebuf)
        pltpu.sync_copy(ebuf, acc_sh.at[idx_v], add=True)
        return carry

    lax.fori_loop(0, per_tile // _CS, chunk, 0)
    plsc.subcore_barrier()
    pltpu.sync_copy(acc_sh.at[pl.ds(s * _NPT, _NPT)],
                    out_hbm.at[c, pl.ds(s * _NPT, _NPT)])


def _scatter(edges_new, idx2):
    mesh = plsc.VectorSubcoreMesh(core_axis_name="c", subcore_axis_name="s",
                                  num_cores=_NC, num_subcores=_NS)
    kfn = pl.kernel(
        _scatter_body,
        out_type=jax.ShapeDtypeStruct((2, _NPAD, _H2), jnp.float32),
        mesh=mesh,
        scratch_types=[
            pltpu.VMEM_SHARED((_NPAD, _H2), jnp.float32),
            pltpu.VMEM((_CS, _H2), jnp.float32),
            pltpu.VMEM((_CS,), jnp.int32),
            pltpu.SemaphoreType.DMA,
        ],
    )
    return kfn(edges_new, idx2)


def _node_body(nodes_ref, sa_ref, ra_ref, wn1a_ref, wn1b_ref, wn1c_ref, bn1_ref,
               wn2_ref, bn2_ref, wg1a_ref, wg1b_ref, bg1_ref, wg2_ref, bg2_ref,
               wg3_ref, bg3_ref, out_ref, s1_acc, ea_acc):
    i = pl.program_id(0)

    @pl.when(i == 0)
    def _init():
        s1_acc[...] = jnp.zeros_like(s1_acc)
        ea_acc[...] = jnp.zeros_like(ea_acc)

    h = jnp.dot(nodes_ref[...], wn1a_ref[...], preferred_element_type=jnp.float32)
    h = h + jnp.dot(sa_ref[...], wn1b_ref[...], preferred_element_type=jnp.float32)
    h = h + jnp.dot(ra_ref[...], wn1c_ref[...], preferred_element_type=jnp.float32)
    h = jnp.maximum(h + bn1_ref[...], 0.0)
    s1_acc[...] = s1_acc[...] + jnp.sum(h, axis=0, keepdims=True)
    ea_acc[...] = ea_acc[...] + jnp.sum(sa_ref[...], axis=0, keepdims=True)

    @pl.when(i == _NB - 1)
    def _final():
        node_agg = jnp.dot(s1_acc[...], wn2_ref[...],
                           preferred_element_type=jnp.float32)
        node_agg = node_agg + jnp.float32(_N) * bn2_ref[...]
        edge_agg = ea_acc[...]
        hg = jnp.dot(node_agg, wg1a_ref[...], preferred_element_type=jnp.float32)
        hg = hg + jnp.dot(edge_agg, wg1b_ref[...], preferred_element_type=jnp.float32)
        hg = jnp.maximum(hg + bg1_ref[...], 0.0)
        hg2 = jnp.dot(hg, wg2_ref[...], preferred_element_type=jnp.float32)
        hg2 = jnp.maximum(hg2 + bg2_ref[...], 0.0)
        out_ref[...] = (jnp.dot(hg2, wg3_ref[...], preferred_element_type=jnp.float32)
                        + bg3_ref[...])


def _node_global(nodes, sent_agg, recv_agg, Wn1a, Wn1b, Wn1c, bn1, Wn2, bn2,
                 Wg1a, Wg1b, bg1, Wg2, bg2, Wg3, bg3):
    full = lambda shape: pl.BlockSpec(shape, lambda i: (0, 0))
    return pl.pallas_call(
        _node_body,
        grid=(_NB,),
        in_specs=[
            pl.BlockSpec((_NBLK, _LAT), lambda i: (i, 0)),
            pl.BlockSpec((_NBLK, _H2), lambda i: (i, 0)),
            pl.BlockSpec((_NBLK, _H2), lambda i: (i, 0)),
            full((_LAT, _H1)),
            full((_H2, _H1)),
            full((_H2, _H1)),
            full((1, _H1)),
            full((_H1, _H2)),
            full((1, _H2)),
            full((_H2, _H1)),
            full((_H2, _H1)),
            full((1, _H1)),
            full((_H1, _H2)),
            full((1, _H2)),
            full((_H2, 1)),
            full((1, 1)),
        ],
        out_specs=pl.BlockSpec((1, 1), lambda i: (0, 0)),
        out_shape=jax.ShapeDtypeStruct((1, 1), jnp.float32),
        scratch_shapes=[
            pltpu.VMEM((1, _H1), jnp.float32),
            pltpu.VMEM((1, _H2), jnp.float32),
        ],
    )(nodes, sent_agg, recv_agg, Wn1a, Wn1b, Wn1c, bn1.reshape(1, _H1),
      Wn2, bn2.reshape(1, _H2), Wg1a, Wg1b, bg1.reshape(1, _H1),
      Wg2, bg2.reshape(1, _H2), Wg3, bg3.reshape(1, 1))


def kernel(x, edge_attr, senders, receivers, W_en, b_en, W_ee, b_ee,
           We1, be1, We2, be2, Wn1, bn1, Wn2, bn2,
           Wg1, bg1, Wg2, bg2, Wg3, bg3):
    We1a = We1[:_LAT]
    We1b = We1[_LAT:2 * _LAT]
    We1c = We1[2 * _LAT:3 * _LAT]
    Wn1a = Wn1[:_LAT]
    Wn1b = Wn1[_LAT:2 * _LAT]
    Wn1c = Wn1[2 * _LAT:3 * _LAT]
    Wg1a = Wg1[:_H2]
    Wg1b = Wg1[_H2:2 * _H2]

    nodes, S, R = _prep(x, W_en, b_en, We1b, We1c)
    G = _gather(S, R, senders, receivers)
    edges_new = _edge_mlp(G, edge_attr, W_ee, b_ee, We1a, be1, We2, be2)
    idx2 = jnp.concatenate([senders, receivers])
    aggs = _scatter(edges_new, idx2)
    out = _node_global(nodes, aggs[0, :_N], aggs[1, :_N], Wn1a, Wn1b, Wn1c, bn1, Wn2, bn2,
                       Wg1a, Wg1b, bg1, Wg2, bg2, Wg3, bg3)
    return out
